# Initial kernel scaffold; baseline (speedup 1.0000x reference)
import jax
import jax.numpy as jnp
from jax.experimental import pallas as pl


def kernel(q_xyz, q_feat, kv_xyz, kv_feat, Wp, bp, Wq, bq, Wk, bk, Wv, bv, Wo, bo, g1, be1, g2, be2, W1, b1, W2, b2):
    raise NotImplementedError("write your pallas kernel here")


# speedup vs baseline: 15.8207x; 15.8207x over previous
import functools

import jax
import jax.numpy as jnp
import numpy as np
from jax.experimental import pallas as pl
from jax.experimental.pallas import tpu as pltpu
from jax.experimental.pallas import tpu_sc as plsc

B, N, M, C, H, K = 4, 2048, 4096, 256, 8, 8
RADIUS = 0.2
DH = C // H
FF = 4 * C
KP = K + 1
BN = B * N
BM = B * M


def _proj_kv_body(kvf_ref, kvx_ref, wk_ref, wv_ref, wpk_ref, ck_ref, bv_ref,
                  out_ref):
    kvf = kvf_ref[...]
    kvx = kvx_ref[...]
    kfull = (jnp.dot(kvf, wk_ref[...], preferred_element_type=jnp.float32)
             + jnp.dot(kvx, wpk_ref[...], preferred_element_type=jnp.float32)
             + ck_ref[...])
    v = jnp.dot(kvf, wv_ref[...], preferred_element_type=jnp.float32) + bv_ref[...]
    out_ref[:, :C] = kfull
    out_ref[:, C:] = v


def _proj_kv(kvf2, kvx2, Wk, Wv, Wpk, ck, bv, interpret=False):
    T = 2048
    grid = (BM // T,)
    return pl.pallas_call(
        _proj_kv_body,
        grid=grid,
        in_specs=[
            pl.BlockSpec((T, C), lambda i: (i, 0)),
            pl.BlockSpec((T, 3), lambda i: (i, 0)),
            pl.BlockSpec((C, C), lambda i: (0, 0)),
            pl.BlockSpec((C, C), lambda i: (0, 0)),
            pl.BlockSpec((3, C), lambda i: (0, 0)),
            pl.BlockSpec((1, C), lambda i: (0, 0)),
            pl.BlockSpec((1, C), lambda i: (0, 0)),
        ],
        out_specs=pl.BlockSpec((T, 2 * C), lambda i: (i, 0)),
        out_shape=jax.ShapeDtypeStruct((BM, 2 * C), jnp.float32),
        interpret=interpret,
    )(kvf2, kvx2, Wk, Wv, Wpk, ck, bv)


_TN = 256


def _topk_body(qx_ref, kxt_ref, idx_ref, valid_ref):
    b = pl.program_id(0)
    qx = qx_ref[0]
    kxt = kxt_ref[0]
    qn = jnp.sum(qx * qx, axis=1, keepdims=True)
    kn = jnp.sum(kxt * kxt, axis=0, keepdims=True)
    dot = jnp.dot(qx, kxt, preferred_element_type=jnp.float32)
    d2 = qn + kn - 2.0 * dot
    dist = jnp.sqrt(jnp.maximum(d2, 1e-12))
    work = jnp.where(dist <= RADIUS, dist, jnp.inf)
    iota = jax.lax.broadcasted_iota(jnp.int32, (_TN, M), 1)
    base = b * M
    for k in range(K):
        mval = jnp.min(work, axis=1, keepdims=True)
        amin = jnp.min(jnp.where(work == mval, iota, M), axis=1,
                       keepdims=True)
        idx_ref[0, :, k:k + 1] = amin + base
        valid_ref[0, :, k:k + 1] = jnp.where(mval < jnp.inf, 1.0, 0.0)
        if k + 1 < K:
            work = jnp.where(iota == amin, jnp.inf, work)


def _topk(q_xyz, kxt, interpret=False):
    grid = (B, N // _TN)
    return pl.pallas_call(
        _topk_body,
        grid=grid,
        in_specs=[
            pl.BlockSpec((1, _TN, 3), lambda b, i: (b, i, 0)),
            pl.BlockSpec((1, 3, M), lambda b, i: (b, 0, 0)),
        ],
        out_specs=[
            pl.BlockSpec((1, _TN, K), lambda b, i: (b, i, 0)),
            pl.BlockSpec((1, _TN, K), lambda b, i: (b, i, 0)),
        ],
        out_shape=[
            jax.ShapeDtypeStruct((B, N, K), jnp.int32),
            jax.ShapeDtypeStruct((B, N, K), jnp.float32),
        ],
        interpret=interpret,
    )(q_xyz, kxt)


def _gather_sc(table, idx_flat):
    info = plsc.get_sparse_core_info()
    nw = info.num_cores * info.num_subcores
    R = idx_flat.shape[0]
    D = table.shape[1]
    rows_per_w = R // nw
    CH = 128
    nch = rows_per_w // CH
    mesh = plsc.VectorSubcoreMesh(core_axis_name="c", subcore_axis_name="s")

    @functools.partial(
        pl.kernel, mesh=mesh,
        out_type=jax.ShapeDtypeStruct((R, D), jnp.float32),
        scratch_types=[
            pltpu.VMEM((CH,), jnp.int32),
            pltpu.VMEM((CH, D), jnp.float32),
            pltpu.SemaphoreType.DMA,
        ],
    )
    def gk(table_hbm, idx_hbm, out_hbm, idx_c, rows_v, sem):
        wid = jax.lax.axis_index("s") * info.num_cores + jax.lax.axis_index("c")
        base = wid * rows_per_w

        def body(i, carry):
            off = base + i * CH
            pltpu.sync_copy(idx_hbm.at[pl.ds(off, CH)], idx_c)
            pltpu.async_copy(table_hbm.at[idx_c], rows_v, sem).wait()
            pltpu.sync_copy(rows_v, out_hbm.at[pl.ds(off, CH)])
            return carry

        jax.lax.fori_loop(0, nch, body, 0)

    return gk(table, idx_flat)


_G = 256


def _attn_body(qf_ref, qx_ref, kvg_ref, valid_ref, wq_ref, wk_ref, wv_ref,
               wpq_ref, wpk_ref, cq_ref, ck_ref, bv_ref, wo_ref, bo_ref,
               g1_ref, be1_ref, g2_ref, be2_ref, w1_ref, b1_ref, w2_ref,
               b2_ref, out_ref):
    f32 = jnp.float32
    qf = qf_ref[...]
    qx = qx_ref[...]
    q0 = (jnp.dot(qf, wq_ref[...], preferred_element_type=f32)
          + jnp.dot(qx, wpq_ref[...], preferred_element_type=f32) + cq_ref[...])
    k0 = (jnp.dot(qf, wk_ref[...], preferred_element_type=f32)
          + jnp.dot(qx, wpk_ref[...], preferred_element_type=f32) + ck_ref[...])
    v0 = jnp.dot(qf, wv_ref[...], preferred_element_type=f32) + bv_ref[...]
    kvg = kvg_ref[...]
    kg = kvg[:, :, :C]
    vg = kvg[:, :, C:]
    hb = (jax.lax.broadcasted_iota(jnp.int32, (C, H), 0) // DH
          == jax.lax.broadcasted_iota(jnp.int32, (C, H), 1)).astype(f32)
    hbt = (jax.lax.broadcasted_iota(jnp.int32, (H, C), 0)
           == jax.lax.broadcasted_iota(jnp.int32, (H, C), 1) // DH).astype(f32)
    scale = 1.0 / np.sqrt(DH).astype(np.float32)
    s0 = jnp.dot(q0 * k0, hb, preferred_element_type=f32) * scale
    pj = q0[:, None, :] * kg
    sj = jnp.dot(pj.reshape(_G * K, C), hb,
                 preferred_element_type=f32).reshape(_G, K, H) * scale
    validb = valid_ref[...]
    sj = jnp.where(validb[:, :, None] > 0.0, sj, -1e9)
    s = jnp.concatenate([s0[:, None, :], sj], axis=1)
    mx = jnp.max(s, axis=1, keepdims=True)
    e = jnp.exp(s - mx)
    w = e / jnp.sum(e, axis=1, keepdims=True)
    wf = jnp.dot(w.reshape(_G * KP, H), hbt,
                 preferred_element_type=f32).reshape(_G, KP, C)
    vfull = jnp.concatenate([v0[:, None, :], vg], axis=1)
    out0 = jnp.sum(wf * vfull, axis=1)
    y = jnp.dot(out0, wo_ref[...], preferred_element_type=f32) + bo_ref[...]

    x = qf + y
    mu = jnp.mean(x, axis=-1, keepdims=True)
    var = jnp.mean((x - mu) ** 2, axis=-1, keepdims=True)
    x = (x - mu) / jnp.sqrt(var + 1e-5) * g1_ref[...] + be1_ref[...]

    h1 = jnp.maximum(jnp.dot(x, w1_ref[...], preferred_element_type=f32)
                     + b1_ref[...], 0.0)
    ffv = jnp.dot(h1, w2_ref[...], preferred_element_type=f32) + b2_ref[...]

    x2 = x + ffv
    mu2 = jnp.mean(x2, axis=-1, keepdims=True)
    var2 = jnp.mean((x2 - mu2) ** 2, axis=-1, keepdims=True)
    x2 = (x2 - mu2) / jnp.sqrt(var2 + 1e-5) * g2_ref[...] + be2_ref[...]

    out_ref[...] = x2 + qf


def _attn(qf2, qx2, kvg, valid2, Wq, Wk, Wv, Wpq, Wpk, cq, ck, bv, Wo, bo,
          g1, be1, g2, be2, W1, b1, W2, b2, interpret=False):
    grid = (BN // _G,)
    full = lambda r, c: pl.BlockSpec((r, c), lambda i: (0, 0))
    return pl.pallas_call(
        _attn_body,
        grid=grid,
        in_specs=[
            pl.BlockSpec((_G, C), lambda i: (i, 0)),
            pl.BlockSpec((_G, 3), lambda i: (i, 0)),
            pl.BlockSpec((_G, K, 2 * C), lambda i: (i, 0, 0)),
            pl.BlockSpec((_G, K), lambda i: (i, 0)),
            full(C, C), full(C, C), full(C, C), full(3, C), full(3, C),
            full(1, C), full(1, C), full(1, C), full(C, C), full(1, C),
            full(1, C), full(1, C), full(1, C), full(1, C),
            full(C, FF), full(1, FF), full(FF, C), full(1, C),
        ],
        out_specs=pl.BlockSpec((_G, C), lambda i: (i, 0)),
        out_shape=jax.ShapeDtypeStruct((BN, C), jnp.float32),
        interpret=interpret,
    )(qf2, qx2, kvg, valid2, Wq, Wk, Wv, Wpq, Wpk, cq, ck, bv, Wo, bo,
      g1, be1, g2, be2, W1, b1, W2, b2)


def kernel(q_xyz, q_feat, kv_xyz, kv_feat, Wp, bp, Wq, bq, Wk, bk, Wv, bv,
           Wo, bo, g1, be1, g2, be2, W1, b1, W2, b2):
    Wpk = Wp @ Wk
    Wpq = Wp @ Wq
    ck = (bp @ Wk + bk).reshape(1, C)
    cq = (bp @ Wq + bq).reshape(1, C)
    bv2 = bv.reshape(1, C)

    kvf2 = kv_feat.reshape(BM, C)
    kvx2 = kv_xyz.reshape(BM, 3)
    kxt = kv_xyz.transpose(0, 2, 1)

    table = _proj_kv(kvf2, kvx2, Wk, Wv, Wpk, ck, bv2)
    idxg, valid = _topk(q_xyz, kxt)

    gathered = _gather_sc(table, idxg.reshape(BN * K))
    kvg = gathered.reshape(BN, K, 2 * C)

    out = _attn(q_feat.reshape(BN, C), q_xyz.reshape(BN, 3), kvg,
                valid.reshape(BN, K), Wq, Wk, Wv, Wpq, Wpk, cq, ck, bv2,
                Wo, bo.reshape(1, C), g1.reshape(1, C), be1.reshape(1, C),
                g2.reshape(1, C), be2.reshape(1, C), W1, b1.reshape(1, FF),
                W2, b2.reshape(1, C))
    return out.reshape(B, N, C)
